# baseline (speedup 1.0000x reference)
import functools
import math

import jax
import jax.numpy as jnp
from jax import lax
from jax.experimental import pallas as pl
from jax.experimental.pallas import tpu as pltpu
from jax.experimental.pallas import tpu_sc as plsc

D_MODEL_ = 64
SCALE_ = math.sqrt(D_MODEL_)
NC_, NS_ = 2, 16
NW_ = NC_ * NS_
IDXW_ = 128
NPG_ = 4
GROUP_ = IDXW_ * NPG_
LANES_ = 16


def _make_emb(n_rows: int):
    assert n_rows % (NW_ * GROUP_) == 0
    per_worker = n_rows // NW_
    ngrp = per_worker // GROUP_
    idx_rows_pw = per_worker // IDXW_

    mesh = plsc.VectorSubcoreMesh(core_axis_name="c", subcore_axis_name="s")

    @functools.partial(
        pl.kernel,
        out_type=jax.ShapeDtypeStruct((n_rows, D_MODEL_), jnp.float32),
        mesh=mesh,
        scratch_types=[
            pltpu.VMEM((NPG_, IDXW_), jnp.int32),
            pltpu.VMEM((GROUP_, D_MODEL_), jnp.float32),
            pltpu.SemaphoreType.DMA,
        ],
        compiler_params=pltpu.CompilerParams(use_tc_tiling_on_sc=False),
    )
    def emb(x_hbm, table_hbm, out_hbm, idx_v, rows_v, sem):
        wid = lax.axis_index("s") * NC_ + lax.axis_index("c")
        row0 = wid * idx_rows_pw

        def group(g, carry):
            ib = row0 + g * NPG_
            pltpu.sync_copy(x_hbm.at[pl.ds(ib, NPG_)], idx_v)
            copies = [
                pltpu.async_copy(
                    table_hbm.at[idx_v.at[j]],
                    rows_v.at[pl.ds(j * IDXW_, IDXW_)],
                    sem,
                )
                for j in range(NPG_)
            ]
            for cp in copies:
                cp.wait()

            def scale_row(r, c):
                for jj in range(D_MODEL_ // LANES_):
                    sl = pl.ds(jj * LANES_, LANES_)
                    rows_v[r, sl] = rows_v[r, sl] * SCALE_
                return c

            lax.fori_loop(0, GROUP_, scale_row, 0)
            pltpu.sync_copy(rows_v, out_hbm.at[pl.ds(ib * IDXW_, GROUP_)])
            return carry

        lax.fori_loop(0, ngrp, group, 0)

    return emb


def kernel(x, table):
    b, t = x.shape
    n_rows = b * t
    xr = x.reshape(n_rows // IDXW_, IDXW_).astype(jnp.int32)
    out = _make_emb(n_rows)(xr, table)
    return out.reshape(b, t, D_MODEL_)


# speedup vs baseline: 1.1384x; 1.1384x over previous
import functools
import math

import jax
import jax.numpy as jnp
from jax import lax
from jax.experimental import pallas as pl
from jax.experimental.pallas import tpu as pltpu
from jax.experimental.pallas import tpu_sc as plsc

D_MODEL_ = 64
SCALE_ = math.sqrt(D_MODEL_)
NC_, NS_ = 2, 16
NW_ = NC_ * NS_
IDXW_ = 128
NPG_ = 2
GROUP_ = IDXW_ * NPG_
NBUF_ = 4
LANES_ = 16
GROUP_BYTES_ = GROUP_ * D_MODEL_ * 4


def _make_emb(n_rows: int):
    assert n_rows % (NW_ * GROUP_ * NBUF_) == 0
    per_worker = n_rows // NW_
    ngrp = per_worker // GROUP_
    idx_rows_pw = per_worker // IDXW_

    mesh = plsc.VectorSubcoreMesh(core_axis_name="c", subcore_axis_name="s")

    @functools.partial(
        pl.kernel,
        out_type=jax.ShapeDtypeStruct((n_rows, D_MODEL_), jnp.float32),
        mesh=mesh,
        scratch_types=[
            pltpu.VMEM((idx_rows_pw, IDXW_), jnp.int32),
            *[pltpu.VMEM((GROUP_, D_MODEL_), jnp.float32) for _ in range(NBUF_)],
            *[pltpu.SemaphoreType.DMA for _ in range(NBUF_)],
            *[pltpu.SemaphoreType.DMA for _ in range(NBUF_)],
        ],
        compiler_params=pltpu.CompilerParams(use_tc_tiling_on_sc=False),
    )
    def emb(x_hbm, table_hbm, out_hbm, idx_v, *bufs_and_sems):
        rows = bufs_and_sems[:NBUF_]
        sem_g = bufs_and_sems[NBUF_:2 * NBUF_]
        sem_w = bufs_and_sems[2 * NBUF_:]

        wid = lax.axis_index("s") * NC_ + lax.axis_index("c")
        row0 = wid * per_worker
        irow0 = wid * idx_rows_pw

        pltpu.sync_copy(x_hbm.at[pl.ds(irow0, idx_rows_pw)], idx_v)

        def fire_gathers(g, b):
            for j in range(NPG_):
                pltpu.async_copy(
                    table_hbm.at[idx_v.at[g * NPG_ + j]],
                    rows[b].at[pl.ds(j * IDXW_, IDXW_)],
                    sem_g[b],
                )

        def wait_gathers(b):
            pltpu.make_async_copy(
                out_hbm.at[pl.ds(0, GROUP_)], rows[b], sem_g[b]
            ).wait()

        def fire_writeout(g, b):
            pltpu.async_copy(
                rows[b], out_hbm.at[pl.ds(row0 + g * GROUP_, GROUP_)], sem_w[b]
            )

        def wait_writeout(b):
            pltpu.make_async_copy(
                rows[b], out_hbm.at[pl.ds(0, GROUP_)], sem_w[b]
            ).wait()

        for b in range(NBUF_ - 1):
            fire_gathers(b, b)

        def outer(G, carry):
            for b in range(NBUF_):
                g = G * NBUF_ + b
                wait_gathers(b)

                def scale_row(r, c):
                    for jj in range(D_MODEL_ // LANES_):
                        sl = pl.ds(jj * LANES_, LANES_)
                        rows[b][r, sl] = rows[b][r, sl] * SCALE_
                    return c

                lax.fori_loop(0, GROUP_, scale_row, 0)
                fire_writeout(g, b)

                gnext = g + NBUF_ - 1
                bn = (b + NBUF_ - 1) % NBUF_

                @pl.when(gnext < ngrp)
                def _():
                    @pl.when(g >= 1)
                    def _():
                        wait_writeout(bn)

                    fire_gathers(gnext, bn)

            return carry

        lax.fori_loop(0, ngrp // NBUF_, outer, 0)

        for b in range(NBUF_):
            wait_writeout(b)

    return emb


def kernel(x, table):
    b, t = x.shape
    n_rows = b * t
    xr = x.reshape(n_rows // IDXW_, IDXW_).astype(jnp.int32)
    out = _make_emb(n_rows)(xr, table)
    return out.reshape(b, t, D_MODEL_)
